# baseline (speedup 1.0000x reference)
import functools

import jax
import jax.numpy as jnp
from jax import lax
from jax.experimental import pallas as pl
from jax.experimental.pallas import tpu as pltpu
from jax.experimental.pallas import tpu_sc as plsc

_POS = 65536
_NEG = 262144
_TOTAL = _POS + _NEG
_DIM = 64
_NC = 2
_NS = 16
_NW = _NC * _NS
_LANES = 16
_CH = 128
_POS_W = _POS // _NW
_NEG_W = _NEG // _NW
_PAIRS_W = _POS_W + _NEG_W
_POS_CHUNKS = _POS_W // _CH
_POS_MARGIN = 0.1
_NEG_MARGIN = 1.0

_TC_COLS = 128
_TC_ROWS = _TOTAL * _LANES // _TC_COLS
_TC_BR = 2048
_TC_GRID = _TC_ROWS // _TC_BR
_TC_POS_BLOCKS = _POS * _LANES // _TC_COLS // _TC_BR
_PAIRS_PER_ROW = _TC_COLS // _LANES


def _make_sc_kernel():
    mesh = plsc.VectorSubcoreMesh(
        core_axis_name="c", subcore_axis_name="s", num_cores=_NC,
        num_subcores=_NS)

    @functools.partial(
        pl.kernel,
        out_type=jax.ShapeDtypeStruct((_TC_ROWS, _TC_COLS), jnp.float32),
        mesh=mesh,
        compiler_params=pltpu.CompilerParams(use_tc_tiling_on_sc=False),
        scratch_types=[
            pltpu.VMEM((_PAIRS_W,), jnp.int32),
            pltpu.VMEM((_PAIRS_W,), jnp.int32),
            pltpu.VMEM((_CH, _DIM), jnp.float32),
            pltpu.VMEM((_CH, _DIM), jnp.float32),
            pltpu.VMEM((_CH, _DIM), jnp.float32),
            pltpu.VMEM((_CH, _DIM), jnp.float32),
            pltpu.VMEM((_CH * _LANES // _TC_COLS, _TC_COLS), jnp.float32),
            pltpu.VMEM((_CH * _LANES // _TC_COLS, _TC_COLS), jnp.float32),
            pltpu.SemaphoreType.DMA,
            pltpu.SemaphoreType.DMA,
            pltpu.SemaphoreType.DMA,
            pltpu.SemaphoreType.DMA,
            pltpu.SemaphoreType.DMA,
            pltpu.SemaphoreType.DMA,
        ],
    )
    def sc_kernel(table_hbm, pix0_hbm, pix1_hbm, nix0_hbm, nix1_hbm,
                  out_hbm, idx0_v, idx1_v, rows_a0, rows_b0, rows_a1,
                  rows_b1, sbuf0, sbuf1, sem_a0, sem_b0, sem_a1, sem_b1,
                  osem0, osem1):
        wid = lax.axis_index("s") * _NC + lax.axis_index("c")
        bufs = ((rows_a0, rows_b0, sem_a0, sem_b0),
                (rows_a1, rows_b1, sem_a1, sem_b1))
        sbufs = ((sbuf0, osem0), (sbuf1, osem1))

        pltpu.sync_copy(pix0_hbm.at[pl.ds(wid * _POS_W, _POS_W)],
                        idx0_v.at[pl.ds(0, _POS_W)])
        pltpu.sync_copy(pix1_hbm.at[pl.ds(wid * _POS_W, _POS_W)],
                        idx1_v.at[pl.ds(0, _POS_W)])
        pltpu.sync_copy(nix0_hbm.at[pl.ds(wid * _NEG_W, _NEG_W)],
                        idx0_v.at[pl.ds(_POS_W, _NEG_W)])
        pltpu.sync_copy(nix1_hbm.at[pl.ds(wid * _NEG_W, _NEG_W)],
                        idx1_v.at[pl.ds(_POS_W, _NEG_W)])

        pos_out = wid * _POS_W
        neg_out = _POS + wid * _NEG_W - _POS_W

        def out_row(c):
            return jnp.where(c < _POS_CHUNKS, pos_out + c * _CH,
                             neg_out + c * _CH)

        def start(c, bi):
            ra, rb, sa, sb = bufs[bi]
            pltpu.async_copy(
                table_hbm.at[idx0_v.at[pl.ds(c * _CH, _CH)]], ra, sa)
            pltpu.async_copy(
                table_hbm.at[idx1_v.at[pl.ds(c * _CH, _CH)]], rb, sb)

        def wait(bi):
            ra, rb, sa, sb = bufs[bi]
            pltpu.make_async_copy(
                table_hbm.at[idx0_v.at[pl.ds(0, _CH)]], ra, sa).wait()
            pltpu.make_async_copy(
                table_hbm.at[idx1_v.at[pl.ds(0, _CH)]], rb, sb).wait()

        _SBR = _CH * _LANES // _TC_COLS

        def start_out(c, sbi):
            sb, osem = sbufs[sbi]
            row = out_row(c) >> 3
            pltpu.async_copy(sb, out_hbm.at[pl.ds(row, _SBR)], osem)

        def wait_out(sbi):
            sb, osem = sbufs[sbi]
            pltpu.make_async_copy(
                sb, out_hbm.at[pl.ds(0, _SBR)], osem).wait()

        def compute(bi, sbi):
            ra, rb, _, _ = bufs[bi]
            sb, _ = sbufs[sbi]

            def group_body(g, carry):
                gbase = g * _LANES
                for j in range(_LANES):
                    p = gbase + j
                    s = None
                    for k in range(_DIM // _LANES):
                        va = ra[p, pl.ds(k * _LANES, _LANES)]
                        vb = rb[p, pl.ds(k * _LANES, _LANES)]
                        df = va - vb
                        s = df * df if s is None else s + df * df
                    sb[2 * g + (j >> 3), pl.ds((j & 7) * _LANES, _LANES)] = s
                return carry

            lax.fori_loop(0, _CH // _LANES, group_body, jnp.int32(0))

        nch2 = _PAIRS_W // _CH // 2
        start(0, 0)

        def chunk2_body(cc, carry):
            c0 = 2 * cc
            start(c0 + 1, 1)
            wait(0)

            @pl.when(cc > 0)
            def _():
                wait_out(0)

            compute(0, 0)
            start_out(c0, 0)

            @pl.when(cc + 1 < nch2)
            def _():
                start(c0 + 2, 0)

            wait(1)

            @pl.when(cc > 0)
            def _():
                wait_out(1)

            compute(1, 1)
            start_out(c0 + 1, 1)
            return carry

        lax.fori_loop(0, nch2, chunk2_body, jnp.int32(0))
        wait_out(0)
        wait_out(1)

    return sc_kernel


_SC_KERNEL = _make_sc_kernel()


def _tc_body(x_hbm, out_ref, buf0, buf1, sem0, sem1):
    bufs = ((buf0, sem0), (buf1, sem1))

    def start(b, bi):
        buf, sem = bufs[bi]
        pltpu.make_async_copy(
            x_hbm.at[pl.ds(b * _TC_BR, _TC_BR), :], buf, sem).start()

    def wait(bi):
        buf, sem = bufs[bi]
        pltpu.make_async_copy(
            x_hbm.at[pl.ds(0, _TC_BR), :], buf, sem).wait()

    def comp(b, bi, acc):
        buf, _ = bufs[bi]
        x = buf[...]
        is_pos = b < _TC_POS_BLOCKS
        col = jax.lax.broadcasted_iota(
            jnp.int32, (_TC_COLS, _PAIRS_PER_ROW), 0)
        seg = jax.lax.broadcasted_iota(
            jnp.int32, (_TC_COLS, _PAIRS_PER_ROW), 1)
        m = (col // _LANES == seg).astype(jnp.float32)
        s = jnp.dot(x, m, preferred_element_type=jnp.float32,
                    precision=jax.lax.Precision.HIGHEST) + 1e-12
        d = jnp.sqrt(s)
        t = jnp.where(is_pos,
                      jnp.maximum(d - _POS_MARGIN, 0.0),
                      jnp.maximum(_NEG_MARGIN - d, 0.0))
        return acc + jnp.sum(t * t)

    start(0, 0)

    def body2(bb, acc):
        b0 = 2 * bb
        start(b0 + 1, 1)
        wait(0)
        acc = comp(b0, 0, acc)

        @pl.when(bb + 1 < _TC_GRID // 2)
        def _():
            start(b0 + 2, 0)

        wait(1)
        acc = comp(b0 + 1, 1, acc)
        return acc

    acc = lax.fori_loop(0, _TC_GRID // 2, body2, jnp.float32(0.0))
    out_ref[...] = acc[None, None]


def _tc_reduce(x):
    return pl.pallas_call(
        _tc_body,
        in_specs=[pl.BlockSpec(memory_space=pl.ANY)],
        out_specs=pl.BlockSpec(memory_space=pltpu.MemorySpace.VMEM),
        out_shape=jax.ShapeDtypeStruct((1, 1), jnp.float32),
        scratch_shapes=[
            pltpu.VMEM((_TC_BR, _TC_COLS), jnp.float32),
            pltpu.VMEM((_TC_BR, _TC_COLS), jnp.float32),
            pltpu.SemaphoreType.DMA,
            pltpu.SemaphoreType.DMA,
        ],
    )(x)


def kernel(table, pos_ix0, pos_ix1, neg_ix0, neg_ix1):
    sq = _SC_KERNEL(table,
                    pos_ix0.astype(jnp.int32), pos_ix1.astype(jnp.int32),
                    neg_ix0.astype(jnp.int32), neg_ix1.astype(jnp.int32))
    return _tc_reduce(sq)[0, 0]


# speedup vs baseline: 1.0690x; 1.0690x over previous
import functools

import jax
import jax.numpy as jnp
from jax import lax
from jax.experimental import pallas as pl
from jax.experimental.pallas import tpu as pltpu
from jax.experimental.pallas import tpu_sc as plsc

_POS = 65536
_NEG = 262144
_TOTAL = _POS + _NEG
_DIM = 64
_NC = 2
_NS = 16
_NW = _NC * _NS
_LANES = 16
_CH = 128
_POS_W = _POS // _NW
_NEG_W = _NEG // _NW
_PAIRS_W = _POS_W + _NEG_W
_POS_CHUNKS = _POS_W // _CH
_POS_MARGIN = 0.1
_NEG_MARGIN = 1.0

_TC_COLS = 128
_TC_ROWS = _TOTAL * _LANES // _TC_COLS
_TC_BR = 4096
_TC_GRID = _TC_ROWS // _TC_BR
_TC_POS_BLOCKS = _POS * _LANES // _TC_COLS // _TC_BR
_PAIRS_PER_ROW = _TC_COLS // _LANES


def _make_sc_kernel():
    mesh = plsc.VectorSubcoreMesh(
        core_axis_name="c", subcore_axis_name="s", num_cores=_NC,
        num_subcores=_NS)

    @functools.partial(
        pl.kernel,
        out_type=jax.ShapeDtypeStruct((_TC_ROWS, _TC_COLS), jnp.float32),
        mesh=mesh,
        compiler_params=pltpu.CompilerParams(use_tc_tiling_on_sc=False),
        scratch_types=[
            pltpu.VMEM((_PAIRS_W,), jnp.int32),
            pltpu.VMEM((_PAIRS_W,), jnp.int32),
            pltpu.VMEM((_CH, _DIM), jnp.float32),
            pltpu.VMEM((_CH, _DIM), jnp.float32),
            pltpu.VMEM((_CH, _DIM), jnp.float32),
            pltpu.VMEM((_CH, _DIM), jnp.float32),
            pltpu.VMEM((_CH * _LANES // _TC_COLS, _TC_COLS), jnp.float32),
            pltpu.VMEM((_CH * _LANES // _TC_COLS, _TC_COLS), jnp.float32),
            pltpu.SemaphoreType.DMA,
            pltpu.SemaphoreType.DMA,
            pltpu.SemaphoreType.DMA,
            pltpu.SemaphoreType.DMA,
            pltpu.SemaphoreType.DMA,
            pltpu.SemaphoreType.DMA,
        ],
    )
    def sc_kernel(table_hbm, pix0_hbm, pix1_hbm, nix0_hbm, nix1_hbm,
                  out_hbm, idx0_v, idx1_v, rows_a0, rows_b0, rows_a1,
                  rows_b1, sbuf0, sbuf1, sem_a0, sem_b0, sem_a1, sem_b1,
                  osem0, osem1):
        wid = lax.axis_index("s") * _NC + lax.axis_index("c")
        bufs = ((rows_a0, rows_b0, sem_a0, sem_b0),
                (rows_a1, rows_b1, sem_a1, sem_b1))
        sbufs = ((sbuf0, osem0), (sbuf1, osem1))

        pltpu.sync_copy(pix0_hbm.at[pl.ds(wid * _POS_W, _POS_W)],
                        idx0_v.at[pl.ds(0, _POS_W)])
        pltpu.sync_copy(pix1_hbm.at[pl.ds(wid * _POS_W, _POS_W)],
                        idx1_v.at[pl.ds(0, _POS_W)])
        pltpu.sync_copy(nix0_hbm.at[pl.ds(wid * _NEG_W, _NEG_W)],
                        idx0_v.at[pl.ds(_POS_W, _NEG_W)])
        pltpu.sync_copy(nix1_hbm.at[pl.ds(wid * _NEG_W, _NEG_W)],
                        idx1_v.at[pl.ds(_POS_W, _NEG_W)])

        pos_out = wid * _POS_W
        neg_out = _POS + wid * _NEG_W - _POS_W

        def out_row(c):
            return jnp.where(c < _POS_CHUNKS, pos_out + c * _CH,
                             neg_out + c * _CH)

        def start(c, bi):
            ra, rb, sa, sb = bufs[bi]
            pltpu.async_copy(
                table_hbm.at[idx0_v.at[pl.ds(c * _CH, _CH)]], ra, sa)
            pltpu.async_copy(
                table_hbm.at[idx1_v.at[pl.ds(c * _CH, _CH)]], rb, sb)

        def wait(bi):
            ra, rb, sa, sb = bufs[bi]
            pltpu.make_async_copy(
                table_hbm.at[idx0_v.at[pl.ds(0, _CH)]], ra, sa).wait()
            pltpu.make_async_copy(
                table_hbm.at[idx1_v.at[pl.ds(0, _CH)]], rb, sb).wait()

        _SBR = _CH * _LANES // _TC_COLS

        def start_out(c, sbi):
            sb, osem = sbufs[sbi]
            row = out_row(c) >> 3
            pltpu.async_copy(sb, out_hbm.at[pl.ds(row, _SBR)], osem)

        def wait_out(sbi):
            sb, osem = sbufs[sbi]
            pltpu.make_async_copy(
                sb, out_hbm.at[pl.ds(0, _SBR)], osem).wait()

        def compute(bi, sbi):
            ra, rb, _, _ = bufs[bi]
            sb, _ = sbufs[sbi]

            def group_body(g, carry):
                gbase = g * _LANES
                for j in range(_LANES):
                    p = gbase + j
                    s = None
                    for k in range(_DIM // _LANES):
                        va = ra[p, pl.ds(k * _LANES, _LANES)]
                        vb = rb[p, pl.ds(k * _LANES, _LANES)]
                        df = va - vb
                        s = df * df if s is None else s + df * df
                    sb[2 * g + (j >> 3), pl.ds((j & 7) * _LANES, _LANES)] = s
                return carry

            lax.fori_loop(0, _CH // _LANES, group_body, jnp.int32(0))

        nch2 = _PAIRS_W // _CH // 2
        start(0, 0)

        def chunk2_body(cc, carry):
            c0 = 2 * cc
            start(c0 + 1, 1)
            wait(0)

            @pl.when(cc > 0)
            def _():
                wait_out(0)

            compute(0, 0)
            start_out(c0, 0)

            @pl.when(cc + 1 < nch2)
            def _():
                start(c0 + 2, 0)

            wait(1)

            @pl.when(cc > 0)
            def _():
                wait_out(1)

            compute(1, 1)
            start_out(c0 + 1, 1)
            return carry

        lax.fori_loop(0, nch2, chunk2_body, jnp.int32(0))
        wait_out(0)
        wait_out(1)

    return sc_kernel


_SC_KERNEL = _make_sc_kernel()


def _tc_body(x_ref, out_ref):
    i = pl.program_id(0)
    x = x_ref[...]
    is_pos = i < _TC_POS_BLOCKS
    col = jax.lax.broadcasted_iota(jnp.int32, (_TC_COLS, _PAIRS_PER_ROW), 0)
    seg = jax.lax.broadcasted_iota(jnp.int32, (_TC_COLS, _PAIRS_PER_ROW), 1)
    m = (col // _LANES == seg).astype(jnp.float32)
    s = jnp.dot(x, m, preferred_element_type=jnp.float32,
                precision=jax.lax.Precision.HIGHEST) + 1e-12
    d = jnp.sqrt(s)
    t = jnp.where(is_pos,
                  jnp.maximum(d - _POS_MARGIN, 0.0),
                  jnp.maximum(_NEG_MARGIN - d, 0.0))
    tot = jnp.sum(t * t)
    prev = out_ref[...]
    out_ref[...] = jnp.where(i == 0, tot, prev[0, 0] + tot)[None, None]


def _tc_reduce(x):
    return pl.pallas_call(
        _tc_body,
        grid=(_TC_GRID,),
        in_specs=[pl.BlockSpec((_TC_BR, _TC_COLS), lambda i: (i, 0))],
        out_specs=pl.BlockSpec((1, 1), lambda i: (0, 0)),
        out_shape=jax.ShapeDtypeStruct((1, 1), jnp.float32),
    )(x)


def kernel(table, pos_ix0, pos_ix1, neg_ix0, neg_ix1):
    sq = _SC_KERNEL(table,
                    pos_ix0.astype(jnp.int32), pos_ix1.astype(jnp.int32),
                    neg_ix0.astype(jnp.int32), neg_ix1.astype(jnp.int32))
    return _tc_reduce(sq)[0, 0]
